# baseline (speedup 1.0000x reference)
import functools

import jax
import jax.numpy as jnp
from jax import lax
from jax.experimental import pallas as pl
from jax.experimental.pallas import tpu as pltpu
from jax.experimental.pallas import tpu_sc as plsc

D = 1024
C = 16
NC = 2
NS = 16
NW = NC * NS
L = 16
UNROLL = 8
NXB = 5
NPB = 2
CPR = D // L
NH = 2
CH = C // NH


def _sc_body(x_hbm, pos_hbm, out_hbm, *scratch):
    xv = scratch[0:NXB]
    pv = scratch[NXB:NXB + NPB]
    xs = scratch[NXB + NPB:NXB + NPB + NXB]
    os_ = scratch[NXB + NPB + NXB:NXB + NPB + 2 * NXB]
    ps = scratch[NXB + NPB + 2 * NXB:]

    nbatch, s, _ = x_hbm.shape
    seq_per_w = s // NW
    nchunks = seq_per_w // C
    nitems = nchunks * nbatch

    wid = lax.axis_index("s") * NC + lax.axis_index("c")
    w0 = wid * seq_per_w

    def start_xload(k):
        it, b = divmod(k, nbatch)
        j = k % NXB
        return pltpu.async_copy(
            x_hbm.at[b, pl.ds(w0 + it * C, C)], xv[j], xs[j])

    def start_posload(it):
        j = it % NPB
        return pltpu.async_copy(
            pos_hbm.at[pl.ds(w0 + it * C, C)], pv[j], ps[j])

    pos_loads = {0: start_posload(0)}
    x_loads = {k: start_xload(k) for k in range(min(NXB - 1, nitems))}
    last_stores = [[] for _ in range(NXB)]

    for k in range(nitems):
        it, b = divmod(k, nbatch)
        j = k % NXB
        if b == 0:
            pos_loads.pop(it).wait()
            if it + 1 < nchunks:
                pos_loads[it + 1] = start_posload(it + 1)
        x_loads.pop(k).wait()

        xbuf = xv[j]
        pbuf = pv[it % NPB]

        stores = []
        for h in range(NH):
            @plsc.parallel_loop(0, CH * CPR, step=1, unroll=UNROLL)
            def _(n, _h=h):
                r = _h * CH + lax.shift_right_logical(n, 6)
                c = pl.multiple_of(
                    lax.shift_left(lax.bitwise_and(n, CPR - 1), 4), L)
                plsc.addupdate(xbuf.at[r, pl.ds(c, L)], pbuf[r, pl.ds(c, L)])

            stores.append(pltpu.async_copy(
                xbuf.at[pl.ds(h * CH, CH)],
                out_hbm.at[b, pl.ds(w0 + it * C + h * CH, CH)],
                os_[j]))
            if h == 0:
                n = k + NXB - 1
                if n < nitems:
                    jn = n % NXB
                    for st in last_stores[jn]:
                        st.wait()
                    last_stores[jn] = []
                    x_loads[n] = start_xload(n)
        last_stores[j] = stores

    for stores in last_stores:
        for st in stores:
            st.wait()


def kernel(x, pos_table):
    b, s, d = x.shape

    mesh = plsc.VectorSubcoreMesh(core_axis_name="c", subcore_axis_name="s")
    run = functools.partial(
        pl.kernel,
        mesh=mesh,
        out_type=jax.ShapeDtypeStruct((b, s, d), jnp.float32),
        scratch_types=(
            [pltpu.VMEM((C, D), jnp.float32) for _ in range(NXB)]
            + [pltpu.VMEM((C, D), jnp.float32) for _ in range(NPB)]
            + [pltpu.SemaphoreType.DMA for _ in range(2 * NXB + NPB)]
        ),
    )(_sc_body)
    return run(x, pos_table)
